# baseline (speedup 1.0000x reference)
import functools

import jax
import jax.numpy as jnp
from jax.experimental import pallas as pl
from jax.experimental.pallas import tpu as pltpu


def _se_fused_kernel(x_ref, w1t_ref, w2t_ref, o_ref, *, inv_hw):
    x = x_ref[...]
    pooled = jnp.sum(x, axis=(2, 3), dtype=jnp.float32) * inv_hw
    h = jnp.dot(pooled, w1t_ref[...], preferred_element_type=jnp.float32)
    h = jnp.maximum(h, 0.0)
    logits = jnp.dot(h, w2t_ref[...], preferred_element_type=jnp.float32)
    gate = pl.reciprocal(1.0 + jnp.exp(-logits), approx=True)
    o_ref[...] = x * gate[:, :, None, None]


@functools.partial(jax.jit, static_argnames=("bt",))
def _se_forward(x, w1t, w2t, bt):
    B, C, H, W = x.shape
    Cr = w1t.shape[1]
    out = pl.pallas_call(
        functools.partial(_se_fused_kernel, inv_hw=1.0 / (H * W)),
        out_shape=jax.ShapeDtypeStruct((B, C, H, W), x.dtype),
        grid=(B // bt,),
        in_specs=[
            pl.BlockSpec((bt, C, H, W), lambda b: (b, 0, 0, 0)),
            pl.BlockSpec((C, Cr), lambda b: (0, 0)),
            pl.BlockSpec((Cr, C), lambda b: (0, 0)),
        ],
        out_specs=pl.BlockSpec((bt, C, H, W), lambda b: (b, 0, 0, 0)),
        compiler_params=pltpu.CompilerParams(
            dimension_semantics=("parallel",),
            vmem_limit_bytes=100 << 20,
        ),
    )(x, w1t, w2t)
    return out


def kernel(x, w1, w2):
    return _se_forward(x, w1.T, w2.T, bt=1)
